# Initial kernel scaffold; baseline (speedup 1.0000x reference)
import jax
import jax.numpy as jnp
from jax.experimental import pallas as pl


def kernel(params, x_node_global_ids, edge_index, target_node_idx, target_tx_features):
    raise NotImplementedError("write your pallas kernel here")


# speedup vs baseline: 4.3149x; 4.3149x over previous
import functools

import jax
import jax.numpy as jnp
from jax import lax
from jax.experimental import pallas as pl
from jax.experimental.pallas import tpu as pltpu
from jax.experimental.pallas import tpu_sc as plsc

N = 10000
E = 320000
D = 128
H = 256
T = 4096

NC = 2
NS = 16
CH = 128
EPAD = 327680
IDPAD = 12288
KC = 160
NH = N // NC
NAH = 5376

_mesh = plsc.VectorSubcoreMesh(
    core_axis_name="c", subcore_axis_name="s", num_cores=NC, num_subcores=NS)


@functools.partial(
    pl.kernel,
    out_type=(
        jax.ShapeDtypeStruct((IDPAD, D), jnp.float32),
        jax.ShapeDtypeStruct((NC, NAH, D), jnp.float32),
    ),
    mesh=_mesh,
    scratch_types=(
        pltpu.VMEM((3, CH), jnp.int32),
        pltpu.VMEM((3 * CH, D), jnp.float32),
        pltpu.VMEM((CH, D), jnp.float32),
        pltpu.VMEM((KC, CH), jnp.int32),
        pltpu.VMEM_SHARED((NAH, D), jnp.float32),
        pltpu.SemaphoreType.DMA,
    ),
)
def _sc_gather_deg(emb_h, ids_h, dst_h, ones_h, zacc_h,
                   h0_out, deg_out, idx_v, rows_v, ones_v, didx_v, dacc, sem):
    c = lax.axis_index("c")
    s = lax.axis_index("s")
    wid = c * NS + s
    own = pl.ds(s * (NAH // NS), NAH // NS)
    pltpu.sync_copy(ids_h.at[wid], idx_v)
    for k in range(3):
        pltpu.async_copy(emb_h.at[idx_v.at[k]], rows_v.at[pl.ds(k * CH, CH)],
                         sem).wait()
    pltpu.sync_copy(rows_v, h0_out.at[pl.ds(wid * 3 * CH, 3 * CH)])
    pltpu.sync_copy(zacc_h, dacc.at[own])
    pltpu.sync_copy(ones_h, ones_v)
    pltpu.sync_copy(dst_h.at[c, s], didx_v)
    plsc.subcore_barrier()

    @pl.loop(0, KC)
    def _(k):
        pltpu.sync_copy(ones_v, dacc.at[didx_v.at[k]], add=True)

    plsc.subcore_barrier()
    pltpu.sync_copy(dacc.at[own], deg_out.at[c, own])


def _make_sc_agg(spc):
    @functools.partial(
        pl.kernel,
        out_type=jax.ShapeDtypeStruct((NC, spc, NAH, D), jnp.float32),
        mesh=_mesh,
        scratch_types=(
            pltpu.VMEM((KC, CH), jnp.int32),
            pltpu.VMEM((KC, CH), jnp.int32),
            pltpu.VMEM((2, CH, D), jnp.float32),
            pltpu.VMEM_SHARED((NAH, D), jnp.float32),
            pltpu.SemaphoreType.DMA,
            pltpu.SemaphoreType.DMA,
        ),
    )
    def agg(table_h, src_h, dst_h, zeros_h, out_h, sidx, didx, rows, acc,
            sem0, sem1):
        c = lax.axis_index("c")
        s = lax.axis_index("s")
        own = pl.ds(s * (NAH // NS), NAH // NS)
        pltpu.sync_copy(dst_h.at[c, s], didx)
        pltpu.sync_copy(zeros_h, acc.at[own])
        sems = (sem0, sem1)
        for j in range(spc):
            pltpu.sync_copy(src_h.at[j, s], sidx)
            plsc.subcore_barrier()
            for b in range(2):
                pltpu.async_copy(table_h.at[sidx.at[b]], rows.at[b], sems[b])

            @pl.loop(0, KC, step=2)
            def _(g):
                for b in range(2):
                    k = g + b
                    pltpu.make_async_copy(table_h.at[sidx.at[0]], rows.at[b],
                                          sems[b]).wait()
                    pltpu.sync_copy(rows.at[b], acc.at[didx.at[k]], add=True)

                    @pl.when(k + 2 < KC)
                    def _():
                        pltpu.async_copy(table_h.at[sidx.at[k + 2]],
                                         rows.at[b], sems[b])

            plsc.subcore_barrier()
            pltpu.sync_copy(acc.at[own], out_h.at[c, j, own])
            if j + 1 < spc:
                pltpu.sync_copy(zeros_h, acc.at[own])

    return agg


_sc_agg0 = _make_sc_agg(1)
_sc_agg1 = _make_sc_agg(2)


@functools.partial(
    pl.kernel,
    out_type=jax.ShapeDtypeStruct((2 * T, H), jnp.float32),
    mesh=_mesh,
    scratch_types=(
        pltpu.VMEM((2, CH), jnp.int32),
        pltpu.VMEM((2 * CH, H), jnp.float32),
        pltpu.SemaphoreType.DMA,
    ),
)
def _sc_target_gather(h2_h, tidx_h, out_h, idx_v, rows_v, sem):
    c = lax.axis_index("c")
    s = lax.axis_index("s")
    wid = c * NS + s
    pltpu.sync_copy(tidx_h.at[wid], idx_v)
    for k in range(2):
        pltpu.async_copy(h2_h.at[idx_v.at[k]], rows_v.at[pl.ds(k * CH, CH)],
                         sem).wait()
    pltpu.sync_copy(rows_v, out_h.at[pl.ds(wid * 2 * CH, 2 * CH)])


_G = 10
_CT = N // _G
_GT = 4
_CTT = T // _GT


_DEG_SPEC = pl.BlockSpec((1, _CT, D), lambda i: (i // 5, i % 5, 0))


def _dinv_of(degp):
    return lax.rsqrt(degp[0, :, 0:1] + 1.0)


def _bf(x):
    return x.astype(jnp.bfloat16).astype(jnp.float32)


def _tc_prep(h0, degp, wr, br):
    def body(h0_r, degp_r, wr_r, br_r, xs0_r, res_r):
        dinv = _dinv_of(degp_r[...])
        h0b = _bf(h0_r[...])
        xs0_r[...] = h0b * dinv
        res_r[...] = jnp.dot(h0b, wr_r[...],
                             preferred_element_type=jnp.float32, precision=lax.Precision.HIGHEST) + br_r[...]

    return pl.pallas_call(
        body,
        grid=(_G,),
        in_specs=[
            pl.BlockSpec((_CT, D), lambda i: (i, 0)),
            _DEG_SPEC,
            pl.BlockSpec((D, H), lambda i: (0, 0)),
            pl.BlockSpec((1, H), lambda i: (0, 0)),
        ],
        out_specs=[
            pl.BlockSpec((_CT, D), lambda i: (i, 0)),
            pl.BlockSpec((_CT, H), lambda i: (i, 0)),
        ],
        out_shape=[
            jax.ShapeDtypeStruct((N, D), jnp.float32),
            jax.ShapeDtypeStruct((N, H), jnp.float32),
        ],
    )(h0, degp, wr, br)


def _tc_mm_stats(layer):
    def body(aggp_r, xs_r, degp_r, w_r, b_r, y_r, st_r):
        i = pl.program_id(0)
        dinv = _dinv_of(degp_r[...])
        if layer == 0:
            a = (aggp_r[0, 0] + xs_r[...]) * dinv
        else:
            a = jnp.concatenate([(aggp_r[0, 0] + xs_r[0]) * dinv,
                                 (aggp_r[0, 1] + xs_r[1]) * dinv], axis=1)
        y = jnp.dot(a, w_r[...], preferred_element_type=jnp.float32,
                    precision=lax.Precision.HIGHEST) + b_r[...]
        y_r[...] = y
        ps = jnp.concatenate([jnp.sum(y, axis=0, keepdims=True),
                              jnp.sum(y * y, axis=0, keepdims=True)], axis=0)

        @pl.when(i == 0)
        def _():
            st_r[...] = ps

        @pl.when(i > 0)
        def _():
            st_r[...] = st_r[...] + ps

    din = D if layer == 0 else H
    spc = 1 if layer == 0 else 2
    xs_spec = (pl.BlockSpec((_CT, D), lambda i: (i, 0)) if layer == 0 else
               pl.BlockSpec((NC, _CT, D), lambda i: (0, i, 0)))

    def run(aggp, xs, degp, w, b):
        return pl.pallas_call(
            body,
            grid=(_G,),
            in_specs=[
                pl.BlockSpec((1, spc, _CT, D),
                             lambda i: (i // 5, 0, i % 5, 0)),
                xs_spec,
                _DEG_SPEC,
                pl.BlockSpec((din, H), lambda i: (0, 0)),
                pl.BlockSpec((1, H), lambda i: (0, 0)),
            ],
            out_specs=[
                pl.BlockSpec((_CT, H), lambda i: (i, 0)),
                pl.BlockSpec((2, H), lambda i: (0, 0)),
            ],
            out_shape=[
                jax.ShapeDtypeStruct((N, H), jnp.float32),
                jax.ShapeDtypeStruct((2, H), jnp.float32),
            ],
        )(aggp, xs, degp, w, b)

    return run


_tc_mm0 = _tc_mm_stats(0)
_tc_mm1 = _tc_mm_stats(1)


def _tc_bn_res(emit_xs):
    def body(*refs):
        if emit_xs:
            y_r, st_r, g_r, bb_r, res_r, degp_r, h_r, xs_r = refs
        else:
            y_r, st_r, g_r, bb_r, res_r, degp_r, h_r = refs
        m = st_r[0:1, :] * (1.0 / N)
        v = st_r[1:2, :] * (1.0 / N) - m * m
        yn = (y_r[...] - m) * lax.rsqrt(v + 1e-5) * g_r[...] + bb_r[...]
        h = jnp.maximum(yn, 0.0) + res_r[...]
        h_r[...] = h
        if emit_xs:
            dinv = _dinv_of(degp_r[...])
            xs = _bf(h) * dinv
            xs_r[0] = xs[:, :D]
            xs_r[1] = xs[:, D:]

    out_specs = [pl.BlockSpec((_CT, H), lambda i: (i, 0))]
    out_shape = [jax.ShapeDtypeStruct((N, H), jnp.float32)]
    if emit_xs:
        out_specs.append(pl.BlockSpec((NC, _CT, D), lambda i: (0, i, 0)))
        out_shape.append(jax.ShapeDtypeStruct((NC, N, D), jnp.float32))

    def run(y, st, g, bb, res, degp):
        return pl.pallas_call(
            body,
            grid=(_G,),
            in_specs=[
                pl.BlockSpec((_CT, H), lambda i: (i, 0)),
                pl.BlockSpec((2, H), lambda i: (0, 0)),
                pl.BlockSpec((1, H), lambda i: (0, 0)),
                pl.BlockSpec((1, H), lambda i: (0, 0)),
                pl.BlockSpec((_CT, H), lambda i: (i, 0)),
                _DEG_SPEC,
            ],
            out_specs=out_specs,
            out_shape=out_shape,
        )(y, st, g, bb, res, degp)

    return run


_tc_bn0 = _tc_bn_res(True)
_tc_bn1 = _tc_bn_res(False)


def _tc_cls1(embs, txf, wtx, btx, w1a, w1b, b1):
    def body(e_r, t_r, wtx_r, btx_r, w1a_r, w1b_r, b1_r, y_r, st_r):
        i = pl.program_id(0)
        tx = jnp.maximum(
            jnp.dot(_bf(t_r[...]), wtx_r[...], preferred_element_type=jnp.float32, precision=lax.Precision.HIGHEST)
            + btx_r[...], 0.0)
        y = (jnp.dot(_bf(e_r[...]), w1a_r[...], preferred_element_type=jnp.float32, precision=lax.Precision.HIGHEST)
             + jnp.dot(_bf(tx), w1b_r[...], preferred_element_type=jnp.float32, precision=lax.Precision.HIGHEST)
             + b1_r[...])
        y_r[...] = y
        ps = jnp.concatenate([jnp.sum(y, axis=0, keepdims=True),
                              jnp.sum(y * y, axis=0, keepdims=True)], axis=0)

        @pl.when(i == 0)
        def _():
            st_r[...] = ps

        @pl.when(i > 0)
        def _():
            st_r[...] = st_r[...] + ps

    return pl.pallas_call(
        body,
        grid=(_GT,),
        in_specs=[
            pl.BlockSpec((_CTT, 2 * H), lambda i: (i, 0)),
            pl.BlockSpec((_CTT, 32), lambda i: (i, 0)),
            pl.BlockSpec((32, 64), lambda i: (0, 0)),
            pl.BlockSpec((1, 64), lambda i: (0, 0)),
            pl.BlockSpec((2 * H, H), lambda i: (0, 0)),
            pl.BlockSpec((64, H), lambda i: (0, 0)),
            pl.BlockSpec((1, H), lambda i: (0, 0)),
        ],
        out_specs=[
            pl.BlockSpec((_CTT, H), lambda i: (i, 0)),
            pl.BlockSpec((2, H), lambda i: (0, 0)),
        ],
        out_shape=[
            jax.ShapeDtypeStruct((T, H), jnp.float32),
            jax.ShapeDtypeStruct((2, H), jnp.float32),
        ],
    )(embs, txf, wtx, btx, w1a, w1b, b1)


def _tc_cls2(y, st, g, bb, w2, b2):
    def body(y_r, st_r, g_r, bb_r, w2_r, b2_r, y2_r, st2_r):
        i = pl.program_id(0)
        m = st_r[0:1, :] * (1.0 / T)
        v = st_r[1:2, :] * (1.0 / T) - m * m
        z = jnp.maximum((y_r[...] - m) * lax.rsqrt(v + 1e-5) * g_r[...]
                        + bb_r[...], 0.0)
        y2 = jnp.dot(_bf(z), w2_r[...], preferred_element_type=jnp.float32, precision=lax.Precision.HIGHEST) + b2_r[...]
        y2_r[...] = y2
        ps = jnp.concatenate([jnp.sum(y2, axis=0, keepdims=True),
                              jnp.sum(y2 * y2, axis=0, keepdims=True)], axis=0)

        @pl.when(i == 0)
        def _():
            st2_r[...] = ps

        @pl.when(i > 0)
        def _():
            st2_r[...] = st2_r[...] + ps

    return pl.pallas_call(
        body,
        grid=(_GT,),
        in_specs=[
            pl.BlockSpec((_CTT, H), lambda i: (i, 0)),
            pl.BlockSpec((2, H), lambda i: (0, 0)),
            pl.BlockSpec((1, H), lambda i: (0, 0)),
            pl.BlockSpec((1, H), lambda i: (0, 0)),
            pl.BlockSpec((H, H // 2), lambda i: (0, 0)),
            pl.BlockSpec((1, H // 2), lambda i: (0, 0)),
        ],
        out_specs=[
            pl.BlockSpec((_CTT, H // 2), lambda i: (i, 0)),
            pl.BlockSpec((2, H // 2), lambda i: (0, 0)),
        ],
        out_shape=[
            jax.ShapeDtypeStruct((T, H // 2), jnp.float32),
            jax.ShapeDtypeStruct((2, H // 2), jnp.float32),
        ],
    )(y, st, g, bb, w2, b2)


def _tc_cls3(y2, st2, g2, bb2, w3, b3):
    def body(y_r, st_r, g_r, bb_r, w3_r, b3_r, z_r):
        m = st_r[0:1, :] * (1.0 / T)
        v = st_r[1:2, :] * (1.0 / T) - m * m
        z = jnp.maximum((y_r[...] - m) * lax.rsqrt(v + 1e-5) * g_r[...]
                        + bb_r[...], 0.0)
        z_r[...] = jnp.dot(_bf(z), w3_r[...],
                           preferred_element_type=jnp.float32, precision=lax.Precision.HIGHEST) + b3_r[...]

    return pl.pallas_call(
        body,
        grid=(_GT,),
        in_specs=[
            pl.BlockSpec((_CTT, H // 2), lambda i: (i, 0)),
            pl.BlockSpec((2, H // 2), lambda i: (0, 0)),
            pl.BlockSpec((1, H // 2), lambda i: (0, 0)),
            pl.BlockSpec((1, H // 2), lambda i: (0, 0)),
            pl.BlockSpec((H // 2, 1), lambda i: (0, 0)),
            pl.BlockSpec((1, 1), lambda i: (0, 0)),
        ],
        out_specs=pl.BlockSpec((_CTT, 1), lambda i: (i, 0)),
        out_shape=jax.ShapeDtypeStruct((T, 1), jnp.float32),
    )(y2, st2, g2, bb2, w3, b3)


def kernel(params, x_node_global_ids, edge_index, target_node_idx,
           target_tx_features):
    f32 = jnp.float32
    row = edge_index[0].astype(jnp.int32)
    col = edge_index[1].astype(jnp.int32)
    ids = x_node_global_ids.astype(jnp.int32)

    ids_p = jnp.concatenate(
        [ids, jnp.zeros((IDPAD - N,), jnp.int32)]).reshape(NC * NS, 3, CH)
    rowp = jnp.concatenate([row, jnp.zeros((EPAD - E,), jnp.int32)])
    colp = jnp.concatenate([col, jnp.full((EPAD - E,), N, jnp.int32)])
    row16 = rowp.reshape(NS, KC, CH)
    src0 = row16.reshape(1, NS, KC, CH)
    src1 = jnp.stack([row16, row16 + N])
    dst = jnp.stack([
        jnp.where((colp >= c * NH) & (colp < c * NH + NH), colp - c * NH, NH)
        for c in range(NC)]).reshape(NC, NS, KC, CH)
    tidx = target_node_idx.astype(jnp.int32).reshape(NC * NS, 2, CH)

    ones = jnp.ones((CH, D), f32)
    zacc = jnp.zeros((NAH // NS, D), f32)

    p = params
    b_of = lambda lin: lin["b"].reshape(1, -1)
    wb = lambda w: w.astype(jnp.bfloat16).astype(f32)

    h0p, degp = _sc_gather_deg(p["emb"], ids_p, dst, ones, zacc)
    h0 = h0p[:N]

    xs0, res = _tc_prep(h0, degp, wb(p["res0"]["W"]), b_of(p["res0"]))
    agg0 = _sc_agg0(xs0, src0, dst, zacc)
    y0, st0 = _tc_mm0(agg0, xs0, degp, wb(p["convs"][0]["W"]), b_of(p["convs"][0]))
    h1, xs12 = _tc_bn0(y0, st0, p["bns"][0]["g"].reshape(1, H),
                       p["bns"][0]["b"].reshape(1, H), res, degp)

    agg1 = _sc_agg1(xs12.reshape(2 * N, D), src1, dst, zacc)
    y1, st1 = _tc_mm1(agg1, xs12, degp, wb(p["convs"][1]["W"]), b_of(p["convs"][1]))
    h2 = _tc_bn1(y1, st1, p["bns"][1]["g"].reshape(1, H),
                 p["bns"][1]["b"].reshape(1, H), h1, degp)[0]

    te = _sc_target_gather(h2, tidx)
    embs = te.reshape(T, 2 * H)
    c = p["cls"]
    w1 = c["l1"]["W"]
    y3, st3 = _tc_cls1(embs, target_tx_features, wb(p["tx"]["W"]), b_of(p["tx"]),
                       wb(w1[:2 * H]), wb(w1[2 * H:]), b_of(c["l1"]))
    y4, st4 = _tc_cls2(y3, st3, c["bn1"]["g"].reshape(1, H),
                       c["bn1"]["b"].reshape(1, H), wb(c["l2"]["W"]), b_of(c["l2"]))
    z = _tc_cls3(y4, st4, c["bn2"]["g"].reshape(1, H // 2),
                 c["bn2"]["b"].reshape(1, H // 2), wb(c["l3"]["W"]), b_of(c["l3"]))
    return z
